# baseline (speedup 1.0000x reference)
import jax
import jax.numpy as jnp
from jax.experimental import pallas as pl
from jax.experimental.pallas import tpu as pltpu

_NBLK = 5120
_PS, _PD = 5, 20


def _body(f_ref, d_ref, w1_ref, b1_ref, wc_ref, bc_ref,
          wra_ref, bra_ref, wrd_ref, brd_ref,
          sem_ref, ang_ref, dist_ref):
    f = f_ref[0]
    x = jnp.dot(w1_ref[...], f, preferred_element_type=jnp.float32)
    x = jnp.maximum(x + b1_ref[...], 0.0)
    sem_nl = jax.lax.dot_general(
        x, wc_ref[...], (((0,), (1,)), ((), ())),
        preferred_element_type=jnp.float32) + bc_ref[...]
    sem_3d = sem_nl.reshape(_NBLK // _PS, _PS, 18)
    sem_ref[0] = jnp.concatenate([sem_3d[:, j, :] for j in range(_PS)], axis=1)
    ang = jnp.dot(wra_ref[...], x, preferred_element_type=jnp.float32)
    ang_ref[0] = ang + bra_ref[...]
    reg_nl = jax.lax.dot_general(
        x, wrd_ref[...], (((0,), (1,)), ((), ())),
        preferred_element_type=jnp.float32) + brd_ref[...]
    reg_3d = reg_nl.reshape(_NBLK // _PD, _PD, 6)
    reg_p = jnp.concatenate([reg_3d[:, j, :] for j in range(_PD)], axis=1)
    dist_ref[0] = d_ref[0] + reg_p


def kernel(fused_feats, obj_scores, distance, W1, b1, gamma1, beta1, Wc, bc, Wr, br):
    B, C, N = fused_feats.shape
    NUM_CLS = Wc.shape[0]
    W1f = W1 * gamma1[:, None]
    b1f = (b1 * gamma1 + beta1)[:, None]
    dist_packed = distance.reshape(B, N // _PD, 6 * _PD)
    nb = pl.cdiv(N, _NBLK)

    out_shapes = (
        jax.ShapeDtypeStruct((B, N // _PS, 18 * _PS), jnp.float32),
        jax.ShapeDtypeStruct((B, 1, N), jnp.float32),
        jax.ShapeDtypeStruct((B, N // _PD, 6 * _PD), jnp.float32),
    )
    sem_p, ang, dist_p = pl.pallas_call(
        _body,
        grid=(B, nb),
        in_specs=[
            pl.BlockSpec((1, C, _NBLK), lambda b, n: (b, 0, n)),
            pl.BlockSpec((1, _NBLK // _PD, 6 * _PD), lambda b, n: (b, n, 0)),
            pl.BlockSpec((C, C), lambda b, n: (0, 0)),
            pl.BlockSpec((C, 1), lambda b, n: (0, 0)),
            pl.BlockSpec((NUM_CLS, C), lambda b, n: (0, 0)),
            pl.BlockSpec((1, NUM_CLS), lambda b, n: (0, 0)),
            pl.BlockSpec((1, C), lambda b, n: (0, 0)),
            pl.BlockSpec((1, 1), lambda b, n: (0, 0)),
            pl.BlockSpec((6, C), lambda b, n: (0, 0)),
            pl.BlockSpec((1, 6), lambda b, n: (0, 0)),
        ],
        out_specs=(
            pl.BlockSpec((1, _NBLK // _PS, 18 * _PS), lambda b, n: (b, n, 0)),
            pl.BlockSpec((1, 1, _NBLK), lambda b, n: (b, 0, n)),
            pl.BlockSpec((1, _NBLK // _PD, 6 * _PD), lambda b, n: (b, n, 0)),
        ),
        out_shape=out_shapes,
        compiler_params=pltpu.CompilerParams(dimension_semantics=("parallel", "parallel")),
    )(fused_feats, dist_packed, W1f, b1f, Wc, bc[None, :],
      Wr[0:1], br[0:1][None, :], Wr[1:7], br[None, 1:7])
    return (sem_p.reshape(B, N, NUM_CLS), ang.reshape(B, N),
            dist_p.reshape(B, N, 6), obj_scores)


# speedup vs baseline: 4.3916x; 4.3916x over previous
import jax
import jax.numpy as jnp
from jax.experimental import pallas as pl

_NBLK = 20000


def _body(f_ref, d_ref, w1_ref, b1_ref, wc_ref, bc_ref, wr_ref, br_ref,
          sem_ref, ang_ref, dist_ref):
    f = f_ref[0]
    x = jnp.dot(w1_ref[...], f, preferred_element_type=jnp.float32)
    x = jnp.maximum(x + b1_ref[...], 0.0)
    sem_ref[0] = jnp.dot(wc_ref[...], x, preferred_element_type=jnp.float32) + bc_ref[...]
    reg = jnp.dot(wr_ref[...], x, preferred_element_type=jnp.float32) + br_ref[...]
    ang_ref[0] = reg[0:1]
    dist_ref[0] = d_ref[0] + reg[1:7]


def kernel(fused_feats, obj_scores, distance, W1, b1, gamma1, beta1, Wc, bc, Wr, br):
    B, C, N = fused_feats.shape
    NUM_CLS = Wc.shape[0]
    W1f = W1 * gamma1[:, None]
    b1f = (b1 * gamma1 + beta1)[:, None]
    nb = pl.cdiv(N, _NBLK)

    grid = (B, nb)
    out_shapes = (
        jax.ShapeDtypeStruct((B, NUM_CLS, N), jnp.float32),
        jax.ShapeDtypeStruct((B, 1, N), jnp.float32),
        jax.ShapeDtypeStruct((B, 6, N), jnp.float32),
    )
    sem_cm, ang, dist_cm = pl.pallas_call(
        _body,
        grid=grid,
        in_specs=[
            pl.BlockSpec((1, C, _NBLK), lambda b, n: (b, 0, n)),
            pl.BlockSpec((1, 6, _NBLK), lambda b, n: (b, 0, n)),
            pl.BlockSpec((C, C), lambda b, n: (0, 0)),
            pl.BlockSpec((C, 1), lambda b, n: (0, 0)),
            pl.BlockSpec((NUM_CLS, C), lambda b, n: (0, 0)),
            pl.BlockSpec((NUM_CLS, 1), lambda b, n: (0, 0)),
            pl.BlockSpec((7, C), lambda b, n: (0, 0)),
            pl.BlockSpec((7, 1), lambda b, n: (0, 0)),
        ],
        out_specs=(
            pl.BlockSpec((1, NUM_CLS, _NBLK), lambda b, n: (b, 0, n)),
            pl.BlockSpec((1, 1, _NBLK), lambda b, n: (b, 0, n)),
            pl.BlockSpec((1, 6, _NBLK), lambda b, n: (b, 0, n)),
        ),
        out_shape=out_shapes,
    )(fused_feats, jnp.transpose(distance, (0, 2, 1)), W1f, b1f,
      Wc, bc[:, None], Wr, br[:, None])
    sem = jnp.transpose(sem_cm, (0, 2, 1))
    dist = jnp.transpose(dist_cm, (0, 2, 1))
    return (sem, ang.reshape(B, N), dist, obj_scores)
